# baseline (speedup 1.0000x reference)
import functools

import jax
import jax.numpy as jnp
from jax import lax
from jax.experimental import pallas as pl
from jax.experimental.pallas import tpu as pltpu
from jax.experimental.pallas import tpu_sc as plsc

U = 64
ITER = 8
NP = 20000
NL = 5000
NN = 1000
PLEN = 8
P2L = 32
P2N = 64

NPp = 20480
NLp = 5120
NNp = 1024

NC = 2
NS = 16
NW = NC * NS

PW = NPp // NW
LW = NLp // NW
NNW = NNp // NW

_SC_MESH = plsc.VectorSubcoreMesh(core_axis_name="c", subcore_axis_name="s")


def _wid():
    return lax.axis_index("s") * NC + lax.axis_index("c")


def _al(x):
    return pl.multiple_of(x, 8)


@functools.partial(
    pl.kernel,
    out_type=(
        jax.ShapeDtypeStruct((NPp * 8,), jnp.float32),
        jax.ShapeDtypeStruct((NLp,), jnp.float32),
    ),
    mesh=_SC_MESH,
    compiler_params=pltpu.CompilerParams(use_tc_tiling_on_sc=False),
    scratch_types=[
        pltpu.VMEM((PW * 8,), jnp.int32),
        pltpu.VMEM((PW * 8,), jnp.float32),
        pltpu.VMEM((P2L * LW,), jnp.int32),
        pltpu.VMEM((P2L * LW,), jnp.float32),
        pltpu.VMEM((LW,), jnp.float32),
        pltpu.SemaphoreType.DMA,
        pltpu.SemaphoreType.DMA,
    ],
)
def _sc_setup(flow_hbm, cap_hbm, l2p_hbm, wp2lT_hbm,
              capg_hbm, pgt_hbm,
              idxs, outv, idxt, val, accv, sem, sem2):
    wid = _wid()
    sems = [sem, sem2]

    def elem_gather(tab_hbm, idx_ref, dst_ref, n):
        def fire(k):
            off = _al(k * 128)
            return pltpu.async_copy(tab_hbm.at[idx_ref.at[pl.ds(off, 128)]],
                                    dst_ref.at[pl.ds(off, 128)], sems[k & 1])
        pend = [fire(0), fire(1)]
        for k in range(n):
            pend[k & 1].wait()
            if k + 2 < n:
                pend[k & 1] = fire(k + 2)

    pltpu.sync_copy(l2p_hbm.at[pl.ds(_al(wid * PW * 8), PW * 8)], idxs)
    elem_gather(cap_hbm, idxs, outv, PW * 8 // 128)
    pltpu.sync_copy(outv, capg_hbm.at[pl.ds(_al(wid * PW * 8), PW * 8)])

    pltpu.sync_copy(wp2lT_hbm.at[pl.ds(_al(wid * P2L * LW), P2L * LW)], idxt)
    elem_gather(flow_hbm, idxt, val, P2L * LW // 128)
    for g in range(LW // 16):
        accv[pl.ds(g * 16, 16)] = jnp.zeros((16,), jnp.float32)

    def pgt_j(j, _):
        for g in range(LW // 16):
            sl = pl.ds(g * 16, 16)
            accv[sl] = accv[sl] + val[pl.ds(j * LW + g * 16, 16)]
        return 0

    lax.fori_loop(0, P2L, pgt_j, 0)
    pltpu.sync_copy(accv, pgt_hbm.at[pl.ds(_al(wid * LW), LW)])


@functools.partial(
    pl.kernel,
    out_type=jax.ShapeDtypeStruct((NNp, U), jnp.float32),
    mesh=_SC_MESH,
    compiler_params=pltpu.CompilerParams(use_tc_tiling_on_sc=False),
    scratch_types=[
        pltpu.VMEM((NNW * 8,), jnp.int32),
        pltpu.VMEM((NNW * 8, U), jnp.float32),
        pltpu.VMEM((NNW, U), jnp.float32),
        pltpu.SemaphoreType.DMA,
        pltpu.SemaphoreType.DMA,
    ],
)
def _sc_dlsum(ls_hbm, l2n_hbm, out_hbm, idxv, buf, acc, sem, sem2):
    wid = _wid()
    pltpu.sync_copy(l2n_hbm.at[pl.ds(_al(wid * NNW * 8), NNW * 8)], idxv)
    c0 = pltpu.async_copy(ls_hbm.at[idxv.at[pl.ds(0, 128)]],
                          buf.at[pl.ds(0, 128)], sem)
    c1 = pltpu.async_copy(ls_hbm.at[idxv.at[pl.ds(128, 128)]],
                          buf.at[pl.ds(128, 128)], sem2)
    c0.wait()
    c1.wait()

    def nsum(n, _):
        for c in range(U // 16):
            sl = pl.ds(c * 16, 16)
            a = buf[8 * n, sl]
            for r in range(1, 8):
                a = a + buf[8 * n + r, sl]
            acc[n, sl] = a
        return 0

    lax.fori_loop(0, NNW, nsum, 0)
    pltpu.sync_copy(acc, out_hbm.at[pl.ds(_al(wid * NNW), NNW)])


CH = 256

@functools.partial(
    pl.kernel,
    out_type=jax.ShapeDtypeStruct((NPp * 8, U), jnp.float32),
    mesh=_SC_MESH,
    compiler_params=pltpu.CompilerParams(use_tc_tiling_on_sc=False),
    scratch_types=[
        pltpu.VMEM((PW * 8,), jnp.int32),
        pltpu.VMEM((PW * 8,), jnp.int32),
        pltpu.VMEM((CH, U), jnp.float32),
        pltpu.VMEM((CH, U), jnp.float32),
        pltpu.VMEM((CH, U), jnp.float32),
        pltpu.VMEM((CH, U), jnp.float32),
        pltpu.VMEM((CH, U), jnp.float32),
        pltpu.VMEM((CH, U), jnp.float32),
        pltpu.SemaphoreType.DMA,
        pltpu.SemaphoreType.DMA,
        pltpu.SemaphoreType.DMA,
        pltpu.SemaphoreType.DMA,
        pltpu.SemaphoreType.DMA,
        pltpu.SemaphoreType.DMA,
    ],
)
def _sc_seqgather(ls_hbm, ds_hbm, wl2p_hbm, wn2p_hbm, seq_hbm,
                  idxa, idxb, ba0, ba1, ba2, bb0, bb1, bb2,
                  sa0, sa1, sa2, sb0, sb1, sb2):
    wid = _wid()
    base = _al(wid * PW * 8)
    pltpu.sync_copy(wl2p_hbm.at[pl.ds(base, PW * 8)], idxa)
    pltpu.sync_copy(wn2p_hbm.at[pl.ds(base, PW * 8)], idxb)
    ba, bb = [ba0, ba1, ba2], [bb0, bb1, bb2]
    sa, sb = [sa0, sa1, sa2], [sb0, sb1, sb2]
    nk = PW * 8 // CH

    def fire(k):
        r = pl.ds(_al(k * CH), CH)
        b = k % 3
        return (pltpu.async_copy(ls_hbm.at[idxa.at[r]], ba[b], sa[b]),
                pltpu.async_copy(ds_hbm.at[idxb.at[r]], bb[b], sb[b]))

    pend = [fire(0), fire(1), fire(2)]
    for k in range(nk):
        b = k % 3
        pend[b][0].wait()
        pend[b][1].wait()
        A, B = ba[b], bb[b]

        def radd(r, _, A=A, B=B):
            for rr in range(4):
                for c in range(U // 16):
                    sl = pl.ds(c * 16, 16)
                    A[4 * r + rr, sl] = A[4 * r + rr, sl] + B[4 * r + rr, sl]
            return 0

        lax.fori_loop(0, CH // 4, radd, 0)
        pltpu.sync_copy(A, seq_hbm.at[pl.ds(_al(wid * PW * 8 + k * CH), CH)])
        if k + 3 < nk:
            pend[b] = fire(k + 3)


@functools.partial(
    pl.kernel,
    out_type=(
        jax.ShapeDtypeStruct((NLp, U), jnp.float32),
        jax.ShapeDtypeStruct((NNp, U), jnp.float32),
    ),
    mesh=_SC_MESH,
    compiler_params=pltpu.CompilerParams(use_tc_tiling_on_sc=False),
    scratch_types=[
        pltpu.VMEM((LW * P2L,), jnp.int32),
        pltpu.VMEM((LW * P2L,), jnp.int32),
        pltpu.VMEM((LW * P2L,), jnp.int32),
        pltpu.VMEM((CH, U), jnp.float32),
        pltpu.VMEM((CH, U), jnp.float32),
        pltpu.VMEM((CH, U), jnp.float32),
        pltpu.VMEM((LW, U), jnp.float32),
        pltpu.VMEM((NNW, U), jnp.float32),
        pltpu.SemaphoreType.DMA,
        pltpu.SemaphoreType.DMA,
        pltpu.SemaphoreType.DMA,
    ],
)
def _sc_agg(pss_hbm, wp2lp_hbm, wp2ls_hbm, wp2np_hbm, wp2ns_hbm,
            la_hbm, na_hbm,
            lp, lsq, lfl, b0, b1, b2, outl, outn, s0, s1, s2):
    wid = _wid()
    z = jnp.zeros((16,), jnp.float32)
    bufs, sems = [b0, b1, b2], [s0, s1, s2]

    def flat_ids(n16):
        def cvt(g, _):
            sl = pl.ds(g * 16, 16)
            lfl[sl] = lp[sl] * 9 + lsq[sl]
            return 0
        lax.fori_loop(0, n16, cvt, 0)

    def fire(k):
        return pltpu.async_copy(
            pss_hbm.at[lfl.at[pl.ds(_al(k * CH), CH)]], bufs[k % 3],
            sems[k % 3])

    base = _al(wid * LW * P2L)
    pltpu.sync_copy(wp2lp_hbm.at[pl.ds(base, LW * P2L)], lp)
    pltpu.sync_copy(wp2ls_hbm.at[pl.ds(base, LW * P2L)], lsq)
    flat_ids(LW * P2L // 16)
    nk = LW * P2L // CH
    pend = [fire(0), fire(1), fire(2)]
    for k in range(nk):
        b = k % 3
        pend[b].wait()
        B = bufs[b]
        for h in range(2):

            def rsum(r, carry, B=B, h=h):
                accs = list(carry)
                for l4 in range(4):
                    for c in range(U // 16):
                        accs[l4 * 4 + c] = accs[l4 * 4 + c] \
                            + B[h * 128 + l4 * 32 + r, pl.ds(c * 16, 16)]
                return tuple(accs)

            a = lax.fori_loop(0, P2L, rsum, (z,) * 16)
            for l4 in range(4):
                for c in range(U // 16):
                    outl[k * 8 + h * 4 + l4, pl.ds(c * 16, 16)] = a[l4 * 4 + c]
        if k + 3 < nk:
            pend[b] = fire(k + 3)
    pltpu.sync_copy(outl, la_hbm.at[pl.ds(_al(wid * LW), LW)])

    nbase = _al(wid * NNW * P2N)
    pltpu.sync_copy(wp2np_hbm.at[pl.ds(nbase, NNW * P2N)],
                    lp.at[pl.ds(0, NNW * P2N)])
    pltpu.sync_copy(wp2ns_hbm.at[pl.ds(nbase, NNW * P2N)],
                    lsq.at[pl.ds(0, NNW * P2N)])
    flat_ids(NNW * P2N // 16)
    nkn = NNW * P2N // CH
    pend = [fire(0), fire(1), fire(2)]
    for k in range(nkn):
        b = k % 3
        pend[b].wait()
        B = bufs[b]

        def rsumn(r, carry, B=B):
            accs = list(carry)
            for n4 in range(4):
                for c in range(U // 16):
                    accs[n4 * 4 + c] = accs[n4 * 4 + c] \
                        + B[n4 * 64 + r, pl.ds(c * 16, 16)]
            return tuple(accs)

        a = lax.fori_loop(0, P2N, rsumn, (z,) * 16)
        for n4 in range(4):
            for c in range(U // 16):
                outn[k * 4 + n4, pl.ds(c * 16, 16)] = a[n4 * 4 + c]
        if k + 3 < nkn:
            pend[b] = fire(k + 3)
    pltpu.sync_copy(outn, na_hbm.at[pl.ds(_al(wid * NNW), NNW)])


def _dot(a, b):
    return jnp.dot(a.astype(jnp.bfloat16), b.astype(jnp.bfloat16),
                   preferred_element_type=jnp.float32)


def _b16(x):
    return x.astype(jnp.bfloat16).astype(jnp.float32)


def _gru_math(x, h, W, Ur, bi, bh):
    xs = _dot(x, W) + bi
    hs = _dot(h, Ur) + bh
    z = jax.nn.sigmoid(xs[:, :U] + hs[:, :U])
    r = jax.nn.sigmoid(xs[:, U:2 * U] + hs[:, U:2 * U])
    hc = jnp.tanh(xs[:, 2 * U:] + r * hs[:, 2 * U:])
    return z * h + (1.0 - z) * hc


def _full(shape):
    return pl.BlockSpec(shape, lambda *_: tuple(0 for _ in shape))


def _tc_enc_path(ft_ref, fp_ref, fps_ref, w1_ref, b1_ref, w2_ref, b2_ref, o_ref):
    w1 = _b16(w1_ref[...])
    x1 = _b16(ft_ref[...] * 1e-3) * w1[0:1, :] + _b16(fp_ref[...] * 1e-2) * w1[1:2, :] \
        + _b16(fps_ref[...] * 1e-3) * w1[2:3, :] + b1_ref[...]
    h = jax.nn.relu(x1)
    o_ref[...] = jax.nn.relu(_dot(h, w2_ref[...]) + b2_ref[...])


def _tc_enc_link(cap_ref, ldt_ref, pgt_ref, w1_ref, b1_ref, w2_ref, b2_ref, o_ref):
    w1 = _b16(w1_ref[...])
    cap = cap_ref[...]
    load = pgt_ref[...] / (cap * 1e9)
    oh = (ldt_ref[...] == 0).astype(jnp.float32)
    x1 = _b16(cap * 1e-2) * w1[0:1, :] + _b16(load) * w1[1:2, :] \
        + oh * w1[2:3, :] + b1_ref[...]
    h = jax.nn.relu(x1)
    o_ref[...] = jax.nn.relu(_dot(h, w2_ref[...]) + b2_ref[...])


def _tc_enc_dev(dl_ref, nod_ref, w1_ref, b1_ref, w2_ref, b2_ref, o_ref):
    w1 = _b16(w1_ref[...])
    dlm = jnp.sum(dl_ref[...], axis=1, keepdims=True) * (1.0 / U)
    dev = (nod_ref[...] == 0).astype(jnp.float32)
    x1 = dev * w1[0:1, :] + _b16(dlm) * w1[1:2, :] + b1_ref[...]
    h = jax.nn.relu(x1)
    o_ref[...] = jax.nn.relu(_dot(h, w2_ref[...]) + b2_ref[...])


def _tc_scan(seq_ref, h0_ref, w_ref, u_ref, bi_ref, bh_ref, o_ref):
    W, Ur, bi, bh = w_ref[...], u_ref[...], bi_ref[...], bh_ref[...]
    h = h0_ref[...]
    o_ref[:, 0, :] = h
    for t in range(8):
        h = _gru_math(seq_ref[:, t, :], h, W, Ur, bi, bh)
        o_ref[:, t + 1, :] = h


def _tc_dualgru(la_ref, lh_ref, lw_ref, lu_ref, lbi_ref, lbh_ref,
                na_ref, nh_ref, dw_ref, du_ref, dbi_ref, dbh_ref,
                lo_ref, no_ref):
    lo_ref[...] = _gru_math(la_ref[...], lh_ref[...], lw_ref[...], lu_ref[...],
                            lbi_ref[...], lbh_ref[...])
    no_ref[...] = _gru_math(na_ref[...], nh_ref[...], dw_ref[...], du_ref[...],
                            dbi_ref[...], dbh_ref[...])


def _tc_readout(pss_ref, capg_ref, w1_ref, b1_ref, w2_ref, b2_ref,
                w3_ref, b3_ref, o_ref):
    w1, b1 = w1_ref[...], b1_ref[...]
    w2, b2 = w2_ref[...], b2_ref[...]
    w3, b3 = w3_ref[...], b3_ref[...]
    acc = jnp.zeros(o_ref.shape, jnp.float32)
    for t in range(1, 9):
        x = pss_ref[:, t, :]
        h1 = jax.nn.relu(_dot(x, w1) + b1)
        h2 = jax.nn.relu(_dot(h1, w2) + b2)
        occ = jax.nn.softplus(_dot(h2, w3) + b3)
        acc = acc + occ / capg_ref[:, t - 1:t]
    o_ref[...] = acc


def _pad_rows(x, n):
    return jnp.pad(x, ((0, n - x.shape[0]),) + ((0, 0),) * (x.ndim - 1))


def kernel(flow_traffic, flow_packets, flow_packet_size, link_capacity,
           link_to_path, path_to_link, nodes, link_to_node, link_device_type,
           node_to_link, node_to_path, path_to_node, params):
    p = params
    f32 = jnp.float32

    ft = _pad_rows(flow_traffic, NPp)
    fp = _pad_rows(flow_packets, NPp)
    fps = _pad_rows(flow_packet_size, NPp)
    cap = jnp.pad(link_capacity, ((0, NLp - NL), (0, 0)), constant_values=1.0)
    ldt = _pad_rows(link_device_type[:, None], NLp)
    nod = _pad_rows(nodes[:, None], NNp)

    l2p_pad = _pad_rows(link_to_path, NPp)
    l2p_flat = l2p_pad.reshape(-1)
    wl2p = l2p_pad.reshape(-1)
    wn2p = _pad_rows(node_to_path, NPp).reshape(-1)
    wp2lT = jnp.pad(path_to_link[:, :, 0], ((0, NLp - NL), (0, 0))).T \
        .reshape(P2L, NW, LW).transpose(1, 0, 2).reshape(-1)
    wp2lp = jnp.pad(path_to_link[:, :, 0], ((0, NLp - NL), (0, 0))).reshape(-1)
    wp2ls = jnp.pad(path_to_link[:, :, 1], ((0, NLp - NL), (0, 0))).reshape(-1)
    wp2np = jnp.pad(path_to_node[:, :, 0], ((0, NNp - NN), (0, 0))).reshape(-1)
    wp2ns = jnp.pad(path_to_node[:, :, 1], ((0, NNp - NN), (0, 0))).reshape(-1)
    wl2n = _pad_rows(link_to_node, NNp).reshape(-1)

    flow1d = jnp.pad(flow_traffic[:, 0], (0, NPp - NP))
    cap1d = jnp.pad(link_capacity[:, 0], (0, NLp - NL))

    b = lambda v: v.reshape(1, -1)

    capg_r, pgt = _sc_setup(flow1d, cap1d, l2p_flat, wp2lT)
    capg = capg_r.reshape(NPp, 8)
    del l2p_pad

    path_state = pl.pallas_call(
        _tc_enc_path,
        grid=(NPp // 4096,),
        in_specs=[
            pl.BlockSpec((4096, 1), lambda i: (i, 0)),
            pl.BlockSpec((4096, 1), lambda i: (i, 0)),
            pl.BlockSpec((4096, 1), lambda i: (i, 0)),
            _full((3, U)), _full((1, U)), _full((U, U)), _full((1, U)),
        ],
        out_specs=pl.BlockSpec((4096, U), lambda i: (i, 0)),
        out_shape=jax.ShapeDtypeStruct((NPp, U), f32),
    )(ft, fp, fps, p['pe_W1'], b(p['pe_b1']), p['pe_W2'], b(p['pe_b2']))

    link_state = pl.pallas_call(
        _tc_enc_link,
        in_specs=[_full((NLp, 1)), _full((NLp, 1)), _full((NLp, 1)),
                  _full((3, U)), _full((1, U)), _full((U, U)), _full((1, U))],
        out_specs=_full((NLp, U)),
        out_shape=jax.ShapeDtypeStruct((NLp, U), f32),
    )(cap, ldt, pgt[:, None], p['le_W1'], b(p['le_b1']), p['le_W2'], b(p['le_b2']))

    dl_sum = _sc_dlsum(link_state, wl2n)

    device_state = pl.pallas_call(
        _tc_enc_dev,
        in_specs=[_full((NNp, U)), _full((NNp, 1)),
                  _full((2, U)), _full((1, U)), _full((U, U)), _full((1, U))],
        out_specs=_full((NNp, U)),
        out_shape=jax.ShapeDtypeStruct((NNp, U), f32),
    )(dl_sum, nod, p['de_W1'], b(p['de_b1']), p['de_W2'], b(p['de_b2']))

    Bp = 2048
    scan_call = pl.pallas_call(
        _tc_scan,
        grid=(NPp // Bp,),
        in_specs=[
            pl.BlockSpec((Bp, 8, U), lambda i: (i, 0, 0)),
            pl.BlockSpec((Bp, U), lambda i: (i, 0)),
            _full((U, 3 * U)), _full((U, 3 * U)),
            _full((1, 3 * U)), _full((1, 3 * U)),
        ],
        out_specs=pl.BlockSpec((Bp, 9, U), lambda i: (i, 0, 0)),
        out_shape=jax.ShapeDtypeStruct((NPp, 9, U), f32),
    )

    dualgru_call = pl.pallas_call(
        _tc_dualgru,
        in_specs=[_full((NLp, U)), _full((NLp, U)),
                  _full((U, 3 * U)), _full((U, 3 * U)),
                  _full((1, 3 * U)), _full((1, 3 * U)),
                  _full((NNp, U)), _full((NNp, U)),
                  _full((U, 3 * U)), _full((U, 3 * U)),
                  _full((1, 3 * U)), _full((1, 3 * U))],
        out_specs=(_full((NLp, U)), _full((NNp, U))),
        out_shape=(jax.ShapeDtypeStruct((NLp, U), f32),
                   jax.ShapeDtypeStruct((NNp, U), f32)),
    )

    pss = None
    for _ in range(ITER):
        seq = _sc_seqgather(link_state, device_state, wl2p, wn2p)
        pss = scan_call(seq.reshape(NPp, 8, U), path_state, p['p_W'], p['p_U'],
                        b(p['p_bi']), b(p['p_bh']))
        path_state = pss[:, 8, :]
        la, na = _sc_agg(pss.reshape(NPp * 9, U), wp2lp, wp2ls, wp2np, wp2ns)
        link_state, device_state = dualgru_call(
            la, link_state, p['l_W'], p['l_U'], b(p['l_bi']), b(p['l_bh']),
            na, device_state, p['d_W'], p['d_U'], b(p['d_bi']), b(p['d_bh']))

    delay = pl.pallas_call(
        _tc_readout,
        grid=(NPp // Bp,),
        in_specs=[
            pl.BlockSpec((Bp, 9, U), lambda i: (i, 0, 0)),
            pl.BlockSpec((Bp, 8), lambda i: (i, 0)),
            _full((U, U // 2)), _full((1, U // 2)),
            _full((U // 2, U // 4)), _full((1, U // 4)),
            _full((U // 4, 1)), _full((1, 1)),
        ],
        out_specs=pl.BlockSpec((Bp, 1), lambda i: (i, 0)),
        out_shape=jax.ShapeDtypeStruct((NPp, 1), f32),
    )(pss, capg, p['ro_W1'], b(p['ro_b1']), p['ro_W2'], b(p['ro_b2']),
      p['ro_W3'], b(p['ro_b3']))

    return delay[:NP]


# speedup vs baseline: 1.1262x; 1.1262x over previous
import functools

import jax
import jax.numpy as jnp
from jax import lax
from jax.experimental import pallas as pl
from jax.experimental.pallas import tpu as pltpu
from jax.experimental.pallas import tpu_sc as plsc

U = 64
ITER = 8
NP = 20000
NL = 5000
NN = 1000
PLEN = 8
P2L = 32
P2N = 64

NPp = 20480
NLp = 5120
NNp = 1024

NC = 2
NS = 16
NW = NC * NS

PW = NPp // NW
LW = NLp // NW
NNW = NNp // NW

_SC_MESH = plsc.VectorSubcoreMesh(core_axis_name="c", subcore_axis_name="s")


def _wid():
    return lax.axis_index("s") * NC + lax.axis_index("c")


def _al(x):
    return pl.multiple_of(x, 8)


@functools.partial(
    pl.kernel,
    out_type=(
        jax.ShapeDtypeStruct((NPp * 8,), jnp.float32),
        jax.ShapeDtypeStruct((NLp,), jnp.float32),
    ),
    mesh=_SC_MESH,
    compiler_params=pltpu.CompilerParams(use_tc_tiling_on_sc=False),
    scratch_types=[
        pltpu.VMEM((PW * 8,), jnp.int32),
        pltpu.VMEM((PW * 8,), jnp.float32),
        pltpu.VMEM((P2L * LW,), jnp.int32),
        pltpu.VMEM((P2L * LW,), jnp.float32),
        pltpu.VMEM((LW,), jnp.float32),
        pltpu.SemaphoreType.DMA,
        pltpu.SemaphoreType.DMA,
    ],
)
def _sc_setup(flow_hbm, cap_hbm, l2p_hbm, wp2lT_hbm,
              capg_hbm, pgt_hbm,
              idxs, outv, idxt, val, accv, sem, sem2):
    wid = _wid()
    sems = [sem, sem2]

    def elem_gather(tab_hbm, idx_ref, dst_ref, n):
        def fire(k):
            off = _al(k * 128)
            return pltpu.async_copy(tab_hbm.at[idx_ref.at[pl.ds(off, 128)]],
                                    dst_ref.at[pl.ds(off, 128)], sems[k & 1])
        pend = [fire(0), fire(1)]
        for k in range(n):
            pend[k & 1].wait()
            if k + 2 < n:
                pend[k & 1] = fire(k + 2)

    pltpu.sync_copy(l2p_hbm.at[pl.ds(_al(wid * PW * 8), PW * 8)], idxs)
    elem_gather(cap_hbm, idxs, outv, PW * 8 // 128)
    pltpu.sync_copy(outv, capg_hbm.at[pl.ds(_al(wid * PW * 8), PW * 8)])

    pltpu.sync_copy(wp2lT_hbm.at[pl.ds(_al(wid * P2L * LW), P2L * LW)], idxt)
    elem_gather(flow_hbm, idxt, val, P2L * LW // 128)
    for g in range(LW // 16):
        accv[pl.ds(g * 16, 16)] = jnp.zeros((16,), jnp.float32)

    def pgt_j(j, _):
        for g in range(LW // 16):
            sl = pl.ds(g * 16, 16)
            accv[sl] = accv[sl] + val[pl.ds(j * LW + g * 16, 16)]
        return 0

    lax.fori_loop(0, P2L, pgt_j, 0)
    pltpu.sync_copy(accv, pgt_hbm.at[pl.ds(_al(wid * LW), LW)])


@functools.partial(
    pl.kernel,
    out_type=jax.ShapeDtypeStruct((NNp, U), jnp.float32),
    mesh=_SC_MESH,
    compiler_params=pltpu.CompilerParams(use_tc_tiling_on_sc=False),
    scratch_types=[
        pltpu.VMEM((NNW * 8,), jnp.int32),
        pltpu.VMEM((NNW * 8, U), jnp.float32),
        pltpu.VMEM((NNW, U), jnp.float32),
        pltpu.SemaphoreType.DMA,
        pltpu.SemaphoreType.DMA,
    ],
)
def _sc_dlsum(ls_hbm, l2n_hbm, out_hbm, idxv, buf, acc, sem, sem2):
    wid = _wid()
    pltpu.sync_copy(l2n_hbm.at[pl.ds(_al(wid * NNW * 8), NNW * 8)], idxv)
    c0 = pltpu.async_copy(ls_hbm.at[idxv.at[pl.ds(0, 128)]],
                          buf.at[pl.ds(0, 128)], sem)
    c1 = pltpu.async_copy(ls_hbm.at[idxv.at[pl.ds(128, 128)]],
                          buf.at[pl.ds(128, 128)], sem2)
    c0.wait()
    c1.wait()

    def nsum(n, _):
        for c in range(U // 16):
            sl = pl.ds(c * 16, 16)
            a = buf[8 * n, sl]
            for r in range(1, 8):
                a = a + buf[8 * n + r, sl]
            acc[n, sl] = a
        return 0

    lax.fori_loop(0, NNW, nsum, 0)
    pltpu.sync_copy(acc, out_hbm.at[pl.ds(_al(wid * NNW), NNW)])


CH = 256

@functools.partial(
    pl.kernel,
    out_type=jax.ShapeDtypeStruct((8, NPp, U), jnp.float32),
    mesh=_SC_MESH,
    compiler_params=pltpu.CompilerParams(use_tc_tiling_on_sc=False),
    scratch_types=[
        pltpu.VMEM((PW * 8,), jnp.int32),
        pltpu.VMEM((PW * 8,), jnp.int32),
        pltpu.VMEM((128, U), jnp.float32),
        pltpu.VMEM((128, U), jnp.float32),
        pltpu.VMEM((128, U), jnp.float32),
        pltpu.VMEM((128, U), jnp.float32),
        pltpu.VMEM((128, U), jnp.float32),
        pltpu.VMEM((128, U), jnp.float32),
        pltpu.SemaphoreType.DMA,
        pltpu.SemaphoreType.DMA,
        pltpu.SemaphoreType.DMA,
        pltpu.SemaphoreType.DMA,
        pltpu.SemaphoreType.DMA,
        pltpu.SemaphoreType.DMA,
    ],
)
def _sc_seqgather(ls_hbm, ds_hbm, wl2p_hbm, wn2p_hbm, seq_hbm,
                  idxa, idxb, ba0, ba1, ba2, bb0, bb1, bb2,
                  sa0, sa1, sa2, sb0, sb1, sb2):
    wid = _wid()
    pltpu.sync_copy(wl2p_hbm.at[pl.ds(_al(wid * PW * 8), PW * 8)], idxa)
    pltpu.sync_copy(wn2p_hbm.at[pl.ds(_al(wid * PW * 8), PW * 8)], idxb)
    ba, bb = [ba0, ba1, ba2], [bb0, bb1, bb2]
    sa, sb = [sa0, sa1, sa2], [sb0, sb1, sb2]
    nk = PW * 8 // 128

    def fire(k):
        r = pl.ds(_al(k * 128), 128)
        b = k % 3
        return (pltpu.async_copy(ls_hbm.at[idxa.at[r]], ba[b], sa[b]),
                pltpu.async_copy(ds_hbm.at[idxb.at[r]], bb[b], sb[b]))

    pend = [fire(0), fire(1), fire(2)]
    for k in range(nk):
        b = k % 3
        pend[b][0].wait()
        pend[b][1].wait()
        A, B = ba[b], bb[b]

        def radd(r, _, A=A, B=B):
            for rr in range(4):
                for c in range(U // 16):
                    sl = pl.ds(c * 16, 16)
                    A[4 * r + rr, sl] = A[4 * r + rr, sl] + B[4 * r + rr, sl]
            return 0

        lax.fori_loop(0, 32, radd, 0)
        t, cc = k // 5, k % 5
        pltpu.sync_copy(A, seq_hbm.at[t, pl.ds(_al(wid * PW + cc * 128), 128)])
        if k + 3 < nk:
            pend[b] = fire(k + 3)


@functools.partial(
    pl.kernel,
    out_type=(
        jax.ShapeDtypeStruct((NLp, U), jnp.float32),
        jax.ShapeDtypeStruct((NNp, U), jnp.float32),
    ),
    mesh=_SC_MESH,
    compiler_params=pltpu.CompilerParams(use_tc_tiling_on_sc=False),
    scratch_types=[
        pltpu.VMEM((LW * P2L,), jnp.int32),
        pltpu.VMEM((LW * P2L,), jnp.int32),
        pltpu.VMEM((LW * P2L,), jnp.int32),
        pltpu.VMEM((CH, U), jnp.float32),
        pltpu.VMEM((CH, U), jnp.float32),
        pltpu.VMEM((CH, U), jnp.float32),
        pltpu.VMEM((LW, U), jnp.float32),
        pltpu.VMEM((NNW, U), jnp.float32),
        pltpu.SemaphoreType.DMA,
        pltpu.SemaphoreType.DMA,
        pltpu.SemaphoreType.DMA,
    ],
)
def _sc_agg(pss_hbm, wp2lp_hbm, wp2ls_hbm, wp2np_hbm, wp2ns_hbm,
            la_hbm, na_hbm,
            lp, lsq, lfl, b0, b1, b2, outl, outn, s0, s1, s2):
    wid = _wid()
    z = jnp.zeros((16,), jnp.float32)
    bufs, sems = [b0, b1, b2], [s0, s1, s2]

    def flat_ids(n16):
        def cvt(g, _):
            sl = pl.ds(g * 16, 16)
            lfl[sl] = lp[sl] * 9 + lsq[sl]
            return 0
        lax.fori_loop(0, n16, cvt, 0)

    def fire(k):
        return pltpu.async_copy(
            pss_hbm.at[lfl.at[pl.ds(_al(k * CH), CH)]], bufs[k % 3],
            sems[k % 3])

    base = _al(wid * LW * P2L)
    pltpu.sync_copy(wp2lp_hbm.at[pl.ds(base, LW * P2L)], lp)
    pltpu.sync_copy(wp2ls_hbm.at[pl.ds(base, LW * P2L)], lsq)
    flat_ids(LW * P2L // 16)
    nk = LW * P2L // CH
    pend = [fire(0), fire(1), fire(2)]
    for k in range(nk):
        b = k % 3
        pend[b].wait()
        B = bufs[b]
        for h in range(2):

            def rsum(r, carry, B=B, h=h):
                accs = list(carry)
                for l4 in range(4):
                    for c in range(U // 16):
                        accs[l4 * 4 + c] = accs[l4 * 4 + c] \
                            + B[h * 128 + l4 * 32 + r, pl.ds(c * 16, 16)]
                return tuple(accs)

            a = lax.fori_loop(0, P2L, rsum, (z,) * 16)
            for l4 in range(4):
                for c in range(U // 16):
                    outl[k * 8 + h * 4 + l4, pl.ds(c * 16, 16)] = a[l4 * 4 + c]
        if k + 3 < nk:
            pend[b] = fire(k + 3)
    pltpu.sync_copy(outl, la_hbm.at[pl.ds(_al(wid * LW), LW)])

    nbase = _al(wid * NNW * P2N)
    pltpu.sync_copy(wp2np_hbm.at[pl.ds(nbase, NNW * P2N)],
                    lp.at[pl.ds(0, NNW * P2N)])
    pltpu.sync_copy(wp2ns_hbm.at[pl.ds(nbase, NNW * P2N)],
                    lsq.at[pl.ds(0, NNW * P2N)])
    flat_ids(NNW * P2N // 16)
    nkn = NNW * P2N // CH
    pend = [fire(0), fire(1), fire(2)]
    for k in range(nkn):
        b = k % 3
        pend[b].wait()
        B = bufs[b]

        def rsumn(r, carry, B=B):
            accs = list(carry)
            for n4 in range(4):
                for c in range(U // 16):
                    accs[n4 * 4 + c] = accs[n4 * 4 + c] \
                        + B[n4 * 64 + r, pl.ds(c * 16, 16)]
            return tuple(accs)

        a = lax.fori_loop(0, P2N, rsumn, (z,) * 16)
        for n4 in range(4):
            for c in range(U // 16):
                outn[k * 4 + n4, pl.ds(c * 16, 16)] = a[n4 * 4 + c]
        if k + 3 < nkn:
            pend[b] = fire(k + 3)
    pltpu.sync_copy(outn, na_hbm.at[pl.ds(_al(wid * NNW), NNW)])


def _dot(a, b):
    return jnp.dot(a.astype(jnp.bfloat16), b.astype(jnp.bfloat16),
                   preferred_element_type=jnp.float32)


def _b16(x):
    return x.astype(jnp.bfloat16).astype(jnp.float32)


def _gru_math(x, h, W, Ur, bi, bh):
    xs = _dot(x, W) + bi
    hs = _dot(h, Ur) + bh
    z = jax.nn.sigmoid(xs[:, :U] + hs[:, :U])
    r = jax.nn.sigmoid(xs[:, U:2 * U] + hs[:, U:2 * U])
    hc = jnp.tanh(xs[:, 2 * U:] + r * hs[:, 2 * U:])
    return z * h + (1.0 - z) * hc


def _full(shape):
    return pl.BlockSpec(shape, lambda *_: tuple(0 for _ in shape))


def _tc_enc_path(ft_ref, fp_ref, fps_ref, w1_ref, b1_ref, w2_ref, b2_ref, o_ref):
    w1 = _b16(w1_ref[...])
    x1 = _b16(ft_ref[...] * 1e-3) * w1[0:1, :] + _b16(fp_ref[...] * 1e-2) * w1[1:2, :] \
        + _b16(fps_ref[...] * 1e-3) * w1[2:3, :] + b1_ref[...]
    h = jax.nn.relu(x1)
    o_ref[...] = jax.nn.relu(_dot(h, w2_ref[...]) + b2_ref[...])


def _tc_enc_link(cap_ref, ldt_ref, pgt_ref, w1_ref, b1_ref, w2_ref, b2_ref, o_ref):
    w1 = _b16(w1_ref[...])
    cap = cap_ref[...]
    load = pgt_ref[...] / (cap * 1e9)
    oh = (ldt_ref[...] == 0).astype(jnp.float32)
    x1 = _b16(cap * 1e-2) * w1[0:1, :] + _b16(load) * w1[1:2, :] \
        + oh * w1[2:3, :] + b1_ref[...]
    h = jax.nn.relu(x1)
    o_ref[...] = jax.nn.relu(_dot(h, w2_ref[...]) + b2_ref[...])


def _tc_enc_dev(dl_ref, nod_ref, w1_ref, b1_ref, w2_ref, b2_ref, o_ref):
    w1 = _b16(w1_ref[...])
    dlm = jnp.sum(dl_ref[...], axis=1, keepdims=True) * (1.0 / U)
    dev = (nod_ref[...] == 0).astype(jnp.float32)
    x1 = dev * w1[0:1, :] + _b16(dlm) * w1[1:2, :] + b1_ref[...]
    h = jax.nn.relu(x1)
    o_ref[...] = jax.nn.relu(_dot(h, w2_ref[...]) + b2_ref[...])


def _tc_scan(seq_ref, h0_ref, w_ref, u_ref, bi_ref, bh_ref, o_ref):
    W, Ur, bi, bh = w_ref[...], u_ref[...], bi_ref[...], bh_ref[...]
    h = h0_ref[...]
    o_ref[:, 0, :] = h
    for t in range(8):
        h = _gru_math(seq_ref[t], h, W, Ur, bi, bh)
        o_ref[:, t + 1, :] = h


def _tc_dualgru(la_ref, lh_ref, lw_ref, lu_ref, lbi_ref, lbh_ref,
                na_ref, nh_ref, dw_ref, du_ref, dbi_ref, dbh_ref,
                lo_ref, no_ref):
    lo_ref[...] = _gru_math(la_ref[...], lh_ref[...], lw_ref[...], lu_ref[...],
                            lbi_ref[...], lbh_ref[...])
    no_ref[...] = _gru_math(na_ref[...], nh_ref[...], dw_ref[...], du_ref[...],
                            dbi_ref[...], dbh_ref[...])


def _tc_readout(pss_ref, capg_ref, w1_ref, b1_ref, w2_ref, b2_ref,
                w3_ref, b3_ref, o_ref):
    w1, b1 = w1_ref[...], b1_ref[...]
    w2, b2 = w2_ref[...], b2_ref[...]
    w3, b3 = w3_ref[...], b3_ref[...]
    acc = jnp.zeros(o_ref.shape, jnp.float32)
    for t in range(1, 9):
        x = pss_ref[:, t, :]
        h1 = jax.nn.relu(_dot(x, w1) + b1)
        h2 = jax.nn.relu(_dot(h1, w2) + b2)
        occ = jax.nn.softplus(_dot(h2, w3) + b3)
        acc = acc + occ / capg_ref[:, t - 1:t]
    o_ref[...] = acc


def _pad_rows(x, n):
    return jnp.pad(x, ((0, n - x.shape[0]),) + ((0, 0),) * (x.ndim - 1))


def kernel(flow_traffic, flow_packets, flow_packet_size, link_capacity,
           link_to_path, path_to_link, nodes, link_to_node, link_device_type,
           node_to_link, node_to_path, path_to_node, params):
    p = params
    f32 = jnp.float32

    ft = _pad_rows(flow_traffic, NPp)
    fp = _pad_rows(flow_packets, NPp)
    fps = _pad_rows(flow_packet_size, NPp)
    cap = jnp.pad(link_capacity, ((0, NLp - NL), (0, 0)), constant_values=1.0)
    ldt = _pad_rows(link_device_type[:, None], NLp)
    nod = _pad_rows(nodes[:, None], NNp)

    l2p_pad = _pad_rows(link_to_path, NPp)
    l2p_flat = l2p_pad.reshape(-1)
    wl2p = l2p_pad.T.reshape(8, NW, PW).transpose(1, 0, 2).reshape(-1)
    wn2p = _pad_rows(node_to_path, NPp).T.reshape(8, NW, PW) \
        .transpose(1, 0, 2).reshape(-1)
    wp2lT = jnp.pad(path_to_link[:, :, 0], ((0, NLp - NL), (0, 0))).T \
        .reshape(P2L, NW, LW).transpose(1, 0, 2).reshape(-1)
    wp2lp = jnp.pad(path_to_link[:, :, 0], ((0, NLp - NL), (0, 0))).reshape(-1)
    wp2ls = jnp.pad(path_to_link[:, :, 1], ((0, NLp - NL), (0, 0))).reshape(-1)
    wp2np = jnp.pad(path_to_node[:, :, 0], ((0, NNp - NN), (0, 0))).reshape(-1)
    wp2ns = jnp.pad(path_to_node[:, :, 1], ((0, NNp - NN), (0, 0))).reshape(-1)
    wl2n = _pad_rows(link_to_node, NNp).reshape(-1)

    flow1d = jnp.pad(flow_traffic[:, 0], (0, NPp - NP))
    cap1d = jnp.pad(link_capacity[:, 0], (0, NLp - NL))

    b = lambda v: v.reshape(1, -1)

    capg_r, pgt = _sc_setup(flow1d, cap1d, l2p_flat, wp2lT)
    capg = capg_r.reshape(NPp, 8)
    del l2p_pad

    path_state = pl.pallas_call(
        _tc_enc_path,
        grid=(NPp // 4096,),
        in_specs=[
            pl.BlockSpec((4096, 1), lambda i: (i, 0)),
            pl.BlockSpec((4096, 1), lambda i: (i, 0)),
            pl.BlockSpec((4096, 1), lambda i: (i, 0)),
            _full((3, U)), _full((1, U)), _full((U, U)), _full((1, U)),
        ],
        out_specs=pl.BlockSpec((4096, U), lambda i: (i, 0)),
        out_shape=jax.ShapeDtypeStruct((NPp, U), f32),
    )(ft, fp, fps, p['pe_W1'], b(p['pe_b1']), p['pe_W2'], b(p['pe_b2']))

    link_state = pl.pallas_call(
        _tc_enc_link,
        in_specs=[_full((NLp, 1)), _full((NLp, 1)), _full((NLp, 1)),
                  _full((3, U)), _full((1, U)), _full((U, U)), _full((1, U))],
        out_specs=_full((NLp, U)),
        out_shape=jax.ShapeDtypeStruct((NLp, U), f32),
    )(cap, ldt, pgt[:, None], p['le_W1'], b(p['le_b1']), p['le_W2'], b(p['le_b2']))

    dl_sum = _sc_dlsum(link_state, wl2n)

    device_state = pl.pallas_call(
        _tc_enc_dev,
        in_specs=[_full((NNp, U)), _full((NNp, 1)),
                  _full((2, U)), _full((1, U)), _full((U, U)), _full((1, U))],
        out_specs=_full((NNp, U)),
        out_shape=jax.ShapeDtypeStruct((NNp, U), f32),
    )(dl_sum, nod, p['de_W1'], b(p['de_b1']), p['de_W2'], b(p['de_b2']))

    Bp = 2048
    scan_call = pl.pallas_call(
        _tc_scan,
        grid=(NPp // Bp,),
        in_specs=[
            pl.BlockSpec((8, Bp, U), lambda i: (0, i, 0)),
            pl.BlockSpec((Bp, U), lambda i: (i, 0)),
            _full((U, 3 * U)), _full((U, 3 * U)),
            _full((1, 3 * U)), _full((1, 3 * U)),
        ],
        out_specs=pl.BlockSpec((Bp, 9, U), lambda i: (i, 0, 0)),
        out_shape=jax.ShapeDtypeStruct((NPp, 9, U), f32),
    )

    dualgru_call = pl.pallas_call(
        _tc_dualgru,
        in_specs=[_full((NLp, U)), _full((NLp, U)),
                  _full((U, 3 * U)), _full((U, 3 * U)),
                  _full((1, 3 * U)), _full((1, 3 * U)),
                  _full((NNp, U)), _full((NNp, U)),
                  _full((U, 3 * U)), _full((U, 3 * U)),
                  _full((1, 3 * U)), _full((1, 3 * U))],
        out_specs=(_full((NLp, U)), _full((NNp, U))),
        out_shape=(jax.ShapeDtypeStruct((NLp, U), f32),
                   jax.ShapeDtypeStruct((NNp, U), f32)),
    )

    pss = None
    for _ in range(ITER):
        seq = _sc_seqgather(link_state, device_state, wl2p, wn2p)
        pss = scan_call(seq, path_state, p['p_W'], p['p_U'],
                        b(p['p_bi']), b(p['p_bh']))
        path_state = pss[:, 8, :]
        la, na = _sc_agg(pss.reshape(NPp * 9, U), wp2lp, wp2ls, wp2np, wp2ns)
        link_state, device_state = dualgru_call(
            la, link_state, p['l_W'], p['l_U'], b(p['l_bi']), b(p['l_bh']),
            na, device_state, p['d_W'], p['d_U'], b(p['d_bi']), b(p['d_bh']))

    delay = pl.pallas_call(
        _tc_readout,
        grid=(NPp // Bp,),
        in_specs=[
            pl.BlockSpec((Bp, 9, U), lambda i: (i, 0, 0)),
            pl.BlockSpec((Bp, 8), lambda i: (i, 0)),
            _full((U, U // 2)), _full((1, U // 2)),
            _full((U // 2, U // 4)), _full((1, U // 4)),
            _full((U // 4, 1)), _full((1, 1)),
        ],
        out_specs=pl.BlockSpec((Bp, 1), lambda i: (i, 0)),
        out_shape=jax.ShapeDtypeStruct((NPp, 1), f32),
    )(pss, capg, p['ro_W1'], b(p['ro_b1']), p['ro_W2'], b(p['ro_b2']),
      p['ro_W3'], b(p['ro_b3']))

    return delay[:NP]
